# baseline (speedup 1.0000x reference)
import jax
import jax.numpy as jnp
from jax.experimental import pallas as pl
from jax.experimental.pallas import tpu as pltpu

B, N, T, E = 8, 1024, 12, 8
IN_DIM, HID, OUT, K = T * E, 64, 32, 3
G = 2


def _fused_kernel(a_ref, x_ref, w1_ref, b1_ref, w2_ref, b2_ref, w3_ref,
                  b3_ref, out_ref):
    row = jax.lax.broadcasted_iota(jnp.int32, (N, N), 0)
    col = jax.lax.broadcasted_iota(jnp.int32, (N, N), 1)
    diag = row == col

    def matmul(p, q):
        return jax.lax.dot_general(
            p, q, (((1,), (0,)), ((), ())),
            preferred_element_type=jnp.float32)

    Ls = []
    for g in range(G):
        a_nd = jnp.where(diag, 0.0, a_ref[g])
        deg = jnp.sum(a_nd, axis=1, keepdims=True)
        dinv = jnp.where(deg > 0, jax.lax.rsqrt(jnp.maximum(deg, 1e-12)),
                         0.0)
        Ls.append(((-dinv * a_nd) * dinv.reshape(1, N)).astype(jnp.bfloat16))

    def cheb(hs, w_ref, b_ref, last):
        w02 = w_ref[0] - w_ref[2]
        outs = []
        for g in range(G):
            hb = hs[g].astype(jnp.bfloat16)
            u = matmul(Ls[g], hb)
            ub = u.astype(jnp.bfloat16)
            v = matmul(ub, w_ref[2])
            o = (matmul(hb, w02) + matmul(ub, w_ref[1])
                 + 2.0 * matmul(Ls[g], v.astype(jnp.bfloat16)) + b_ref[0])
            outs.append(o if last else jnp.maximum(o, 0.0))
        return outs

    hs = [x_ref[g] for g in range(G)]
    hs = cheb(hs, w1_ref, b1_ref, False)
    hs = cheb(hs, w2_ref, b2_ref, False)
    hs = cheb(hs, w3_ref, b3_ref, True)
    for g in range(G):
        out_ref[g] = hs[g]


def kernel(X, A, W1, b1, W2, b2, W3, b3):
    x = X.reshape(B, N, IN_DIM)
    b1r = b1.reshape(1, HID)
    b2r = b2.reshape(1, HID)
    b3r = b3.reshape(1, OUT)

    full = lambda *s: pl.BlockSpec(s, lambda b: (0,) * len(s))
    return pl.pallas_call(
        _fused_kernel,
        grid=(B // G,),
        in_specs=[
            pl.BlockSpec((G, N, N), lambda b: (b, 0, 0)),
            pl.BlockSpec((G, N, IN_DIM), lambda b: (b, 0, 0)),
            full(K, IN_DIM, HID),
            full(1, HID),
            full(K, HID, HID),
            full(1, HID),
            full(K, HID, OUT),
            full(1, OUT),
        ],
        out_specs=pl.BlockSpec((G, N, OUT), lambda b: (b, 0, 0)),
        out_shape=jax.ShapeDtypeStruct((B, N, OUT), jnp.float32),
        compiler_params=pltpu.CompilerParams(
            dimension_semantics=("parallel",),
        ),
    )(A, x, W1, b1r, W2, b2r, W3, b3r)


# speedup vs baseline: 1.0779x; 1.0779x over previous
import jax
import jax.numpy as jnp
from jax.experimental import pallas as pl
from jax.experimental.pallas import tpu as pltpu

B, N, T, E = 8, 1024, 12, 8
IN_DIM, HID, OUT, K = T * E, 64, 32, 3
G = 2


def _fused_kernel(a_ref, x_ref, w1_ref, b1_ref, w2_ref, b2_ref, w3_ref,
                  b3_ref, out_ref):
    row = jax.lax.broadcasted_iota(jnp.int32, (N, N), 0)
    col = jax.lax.broadcasted_iota(jnp.int32, (N, N), 1)
    diag = row == col

    def matmul(p, q):
        return jax.lax.dot_general(
            p, q, (((1,), (0,)), ((), ())),
            preferred_element_type=jnp.float32)

    Ls = []
    for g in range(G):
        a_nd = jnp.where(diag, 0.0, a_ref[g])
        deg = jnp.sum(a_nd, axis=1, keepdims=True)
        dinv = jnp.where(deg > 0, jax.lax.rsqrt(jnp.maximum(deg, 1e-12)),
                         0.0)
        Ls.append(((-dinv * a_nd) * dinv.reshape(1, N)).astype(jnp.bfloat16))

    def cheb(hs, w_ref, b_ref, last):
        w02 = w_ref[0] - w_ref[2]
        hbs = [hs[g].astype(jnp.bfloat16) for g in range(G)]
        us = [matmul(Ls[g], hbs[g]) for g in range(G)]
        ubs = [us[g].astype(jnp.bfloat16) for g in range(G)]
        vs = [matmul(ubs[g], w_ref[2]) for g in range(G)]
        lvs = [matmul(Ls[g], vs[g].astype(jnp.bfloat16)) for g in range(G)]
        outs = []
        for g in range(G):
            o = (matmul(hbs[g], w02) + matmul(ubs[g], w_ref[1])
                 + 2.0 * lvs[g] + b_ref[0])
            outs.append(o if last else jnp.maximum(o, 0.0))
        return outs

    hs = [x_ref[g] for g in range(G)]
    hs = cheb(hs, w1_ref, b1_ref, False)
    hs = cheb(hs, w2_ref, b2_ref, False)
    hs = cheb(hs, w3_ref, b3_ref, True)
    for g in range(G):
        out_ref[g] = hs[g]


def kernel(X, A, W1, b1, W2, b2, W3, b3):
    x = X.reshape(B, N, IN_DIM)
    b1r = b1.reshape(1, HID)
    b2r = b2.reshape(1, HID)
    b3r = b3.reshape(1, OUT)

    full = lambda *s: pl.BlockSpec(s, lambda b: (0,) * len(s))
    return pl.pallas_call(
        _fused_kernel,
        grid=(B // G,),
        in_specs=[
            pl.BlockSpec((G, N, N), lambda b: (b, 0, 0)),
            pl.BlockSpec((G, N, IN_DIM), lambda b: (b, 0, 0)),
            full(K, IN_DIM, HID),
            full(1, HID),
            full(K, HID, HID),
            full(1, HID),
            full(K, HID, OUT),
            full(1, OUT),
        ],
        out_specs=pl.BlockSpec((G, N, OUT), lambda b: (b, 0, 0)),
        out_shape=jax.ShapeDtypeStruct((B, N, OUT), jnp.float32),
        compiler_params=pltpu.CompilerParams(
            dimension_semantics=("parallel",),
        ),
    )(A, x, W1, b1r, W2, b2r, W3, b3r)
